# Initial kernel scaffold; baseline (speedup 1.0000x reference)
import jax
import jax.numpy as jnp
from jax.experimental import pallas as pl


def kernel(token_ids, weight):
    raise NotImplementedError("write your pallas kernel here")


# speedup vs baseline: 1.8446x; 1.8446x over previous
import functools

import jax
import jax.numpy as jnp
from jax import lax
from jax.experimental import pallas as pl
from jax.experimental.pallas import tpu as pltpu
from jax.experimental.pallas import tpu_sc as plsc

_NUM_CORES = 2
_NUM_SUBCORES = 16
_NW = _NUM_CORES * _NUM_SUBCORES

_D = 64
_CHUNK = 1024
_GATHER = 128


def _sc_gather(table, idx_flat, b_per_w, n_chunks):
    mesh = plsc.VectorSubcoreMesh(core_axis_name="c", subcore_axis_name="s")
    B = idx_flat.shape[0]

    @functools.partial(
        pl.kernel,
        out_type=jax.ShapeDtypeStruct((B, _D), jnp.float32),
        mesh=mesh,
        scratch_types=[
            pltpu.VMEM((_CHUNK,), jnp.int32),
            pltpu.VMEM((_CHUNK, _D), jnp.float32),
            pltpu.SemaphoreType.DMA,
        ],
        compiler_params=pltpu.CompilerParams(use_tc_tiling_on_sc=False),
    )
    def k(table_hbm, idx_hbm, out_hbm, idx_v, rows_v, sem):
        wid = lax.axis_index("s") * _NUM_CORES + lax.axis_index("c")
        base = wid * b_per_w

        def body(g, carry):
            off = base + g * _CHUNK
            pltpu.sync_copy(idx_hbm.at[pl.ds(off, _CHUNK)], idx_v)
            copies = []
            for j in range(_CHUNK // _GATHER):
                copies.append(pltpu.async_copy(
                    table_hbm.at[idx_v.at[pl.ds(j * _GATHER, _GATHER)]],
                    rows_v.at[pl.ds(j * _GATHER, _GATHER)],
                    sem,
                ))
            for c in copies:
                c.wait()
            pltpu.sync_copy(rows_v, out_hbm.at[pl.ds(off, _CHUNK)])
            return carry

        lax.fori_loop(0, n_chunks, body, 0)

    return k(table, idx_flat)


def kernel(token_ids, weight):
    S, T = token_ids.shape
    B = S * T
    assert B % (_NW * _CHUNK) == 0
    b_per_w = B // _NW
    n_chunks = b_per_w // _CHUNK
    idx_flat = token_ids.reshape(B).astype(jnp.int32)
    out = _sc_gather(weight, idx_flat, b_per_w, n_chunks)
    return out.reshape(S, T, weight.shape[1])


# speedup vs baseline: 1.8571x; 1.0068x over previous
import functools

import jax
import jax.numpy as jnp
from jax import lax
from jax.experimental import pallas as pl
from jax.experimental.pallas import tpu as pltpu
from jax.experimental.pallas import tpu_sc as plsc

_NUM_CORES = 2
_NUM_SUBCORES = 16
_NW = _NUM_CORES * _NUM_SUBCORES

_D = 64
_CHUNK = 640
_GATHER = 128
_NG = _CHUNK // _GATHER
_NBUF = 2


def _sc_gather(table, idx_flat, b_per_w, n_chunks):
    mesh = plsc.VectorSubcoreMesh(core_axis_name="c", subcore_axis_name="s")
    B = idx_flat.shape[0]
    n_outer = n_chunks // _NBUF

    @functools.partial(
        pl.kernel,
        out_type=jax.ShapeDtypeStruct((B, _D), jnp.float32),
        mesh=mesh,
        scratch_types=(
            [pltpu.VMEM((_CHUNK,), jnp.int32) for _ in range(_NBUF)]
            + [pltpu.VMEM((_CHUNK, _D), jnp.float32) for _ in range(_NBUF)]
            + [pltpu.SemaphoreType.DMA for _ in range(2 * _NBUF)]
        ),
        compiler_params=pltpu.CompilerParams(use_tc_tiling_on_sc=False),
    )
    def k(table_hbm, idx_hbm, out_hbm, *bufs):
        idx_v = bufs[:_NBUF]
        rows_v = bufs[_NBUF:2 * _NBUF]
        gsem = bufs[2 * _NBUF:3 * _NBUF]
        wsem = bufs[3 * _NBUF:]
        wid = lax.axis_index("s") * _NUM_CORES + lax.axis_index("c")
        base = wid * b_per_w

        def fire_gathers(b, g):
            off = base + g * _CHUNK
            pltpu.sync_copy(idx_hbm.at[pl.ds(off, _CHUNK)], idx_v[b])
            for j in range(_NG):
                pltpu.async_copy(
                    table_hbm.at[idx_v[b].at[pl.ds(j * _GATHER, _GATHER)]],
                    rows_v[b].at[pl.ds(j * _GATHER, _GATHER)],
                    gsem[b],
                )

        def wait_gathers(b):
            for j in range(_NG):
                pltpu.make_async_copy(
                    table_hbm.at[idx_v[b].at[pl.ds(j * _GATHER, _GATHER)]],
                    rows_v[b].at[pl.ds(j * _GATHER, _GATHER)],
                    gsem[b],
                ).wait()

        def wait_writeback(b):
            pltpu.make_async_copy(
                rows_v[b], out_hbm.at[pl.ds(base, _CHUNK)], wsem[b]
            ).wait()

        for b in range(_NBUF):
            fire_gathers(b, b)

        def outer(t, carry):
            for b in range(_NBUF):
                g = t * _NBUF + b
                off = base + g * _CHUNK
                wait_gathers(b)
                pltpu.async_copy(rows_v[b], out_hbm.at[pl.ds(off, _CHUNK)],
                                 wsem[b])

                @pl.when(t < n_outer - 1)
                def _():
                    wait_writeback(b)
                    fire_gathers(b, g + _NBUF)
            return carry

        lax.fori_loop(0, n_outer, outer, 0)
        for b in range(_NBUF):
            wait_writeback(b)

    return k(table, idx_flat)


def kernel(token_ids, weight):
    S, T = token_ids.shape
    B = S * T
    b_per_w = B // _NW
    n_chunks = b_per_w // _CHUNK
    assert b_per_w % (_CHUNK * _NBUF) == 0
    idx_flat = token_ids.reshape(B).astype(jnp.int32)
    out = _sc_gather(weight, idx_flat, b_per_w, n_chunks)
    return out.reshape(S, T, weight.shape[1])
